# baseline (speedup 1.0000x reference)
import functools
import math

import jax
import jax.numpy as jnp
from jax import lax
from jax.experimental import pallas as pl
from jax.experimental.pallas import tpu as pltpu
from jax.experimental.pallas import tpu_sc as plsc

_FIELD_CONSTANT = 4.0 * math.pi * 14.399645351950548
_CUBIC_MADELUNG = -2.8372974794806
_CONST = _FIELD_CONSTANT / (4.0 * math.pi)
_PI = math.pi

_N = 1600000
_B = 4096
_NC = 2
_NS = 16
_NW = _NC * _NS
_W = _N // _NW
_CH = 2000
_NCHUNK = _W // _CH
_NVEC = _CH // 16
_ACC = 5 * _B


def _sc_segment_sums(cc, pos, batch):
    mesh = plsc.VectorSubcoreMesh(core_axis_name="c", subcore_axis_name="s")

    @functools.partial(
        pl.kernel,
        out_type=jax.ShapeDtypeStruct((_NW, _ACC), jnp.float32),
        mesh=mesh,
        scratch_types=[
            pltpu.VMEM((_ACC,), jnp.float32),
            pltpu.VMEM((_CH, 4), jnp.float32),
            pltpu.VMEM((_CH, 3), jnp.float32),
            pltpu.VMEM((_CH,), jnp.int32),
        ],
        compiler_params=pltpu.CompilerParams(needs_layout_passes=False,
                                             use_tc_tiling_on_sc=False),
    )
    def body(cc_hbm, pos_hbm, b_hbm, out_hbm, acc, ccb, posb, bb):
        c = lax.axis_index("c")
        s = lax.axis_index("s")
        wid = s * _NC + c
        base_atom = wid * _W
        iota = lax.iota(jnp.int32, 16)
        zeros16f = jnp.zeros((16,), jnp.float32)
        col0 = jnp.zeros((16,), jnp.int32)

        def zero_chunk(i, carry):
            acc[pl.ds(i * 16, 16)] = zeros16f
            return carry
        lax.fori_loop(0, _ACC // 16, zero_chunk, 0)

        def vec_body(vi, carry):
            b0 = vi * 16
            rows = b0 + iota
            ids = bb[pl.ds(b0, 16)]
            q = plsc.load_gather(ccb, [rows, col0])
            c1 = plsc.load_gather(ccb, [rows, col0 + 1])
            c2 = plsc.load_gather(ccb, [rows, col0 + 2])
            c3 = plsc.load_gather(ccb, [rows, col0 + 3])
            px = plsc.load_gather(posb, [rows, col0])
            py = plsc.load_gather(posb, [rows, col0 + 1])
            pz = plsc.load_gather(posb, [rows, col0 + 2])

            d0 = q * px + c3
            d1 = q * py + c1
            d2 = q * pz + c2
            r2 = px * px + py * py + pz * pz
            pdr = px * c3 + py * c1 + pz * c2
            qq = r2 * q + 2.0 * pdr

            nxt = b0 + jnp.minimum(iota + 1, 15)
            ids_n = plsc.load_gather(bb, [nxt])
            is_bound = ids != ids_n
            is_last = jnp.logical_or(is_bound, iota == 15)

            for k, contrib in enumerate((q, d0, d1, d2, qq)):
                cum = plsc.cumsum(contrib)
                off = jnp.int32(k * _B)
                plsc.addupdate_scatter(acc, [ids + off], cum, mask=is_last)
                plsc.addupdate_scatter(acc, [ids_n + off], -cum,
                                       mask=is_bound)
            return carry

        def chunk_body(ci, carry):
            start = base_atom + ci * _CH
            pltpu.sync_copy(cc_hbm.at[pl.ds(start, _CH)], ccb)
            pltpu.sync_copy(pos_hbm.at[pl.ds(start, _CH)], posb)
            pltpu.sync_copy(b_hbm.at[pl.ds(start, _CH)], bb)
            return lax.fori_loop(0, _NVEC, vec_body, carry)

        lax.fori_loop(0, _NCHUNK, chunk_body, 0)
        pltpu.sync_copy(acc, out_hbm.at[wid])

    return body(cc, pos, batch)


def _tc_combine_body(p_ref, v_ref, o_ref):
    p = jnp.sum(p_ref[...], axis=0)
    t = p[0 * _B:1 * _B]
    d0 = p[1 * _B:2 * _B]
    d1 = p[2 * _B:3 * _B]
    d2 = p[3 * _B:4 * _B]
    quad = p[4 * _B:5 * _B]
    vol = v_ref[...]
    ls = jnp.power(vol, 0.3333)
    de = 0.5 * _CUBIC_MADELUNG * _CONST * t * t / ls
    de = de + 2.0 * _CONST * _PI * (d0 * d0 + d1 * d1 + d2 * d2) / (3.0 * vol)
    de = de - 2.0 * _CONST * _PI * t * quad / (3.0 * vol)
    o_ref[...] = de


def kernel(charge_coefficients, positions, volumes, batch):
    batch_i = batch.astype(jnp.int32)
    partials = _sc_segment_sums(charge_coefficients, positions, batch_i)
    de = pl.pallas_call(
        _tc_combine_body,
        out_shape=jax.ShapeDtypeStruct((_B,), jnp.float32),
    )(partials, volumes)
    return de


# speedup vs baseline: 20.2516x; 20.2516x over previous
import functools
import math

import jax
import jax.numpy as jnp
from jax import lax
from jax.experimental import pallas as pl
from jax.experimental.pallas import tpu as pltpu
from jax.experimental.pallas import tpu_sc as plsc

_FIELD_CONSTANT = 4.0 * math.pi * 14.399645351950548
_CUBIC_MADELUNG = -2.8372974794806
_CONST = _FIELD_CONSTANT / (4.0 * math.pi)
_PI = math.pi

_N = 1600000
_B = 4096
_NC = 2
_NS = 16
_NW = _NC * _NS
_W = _N // _NW
_CH = 2000
_NCHUNK = _W // _CH
_NVEC = _CH // 16
_ACC = 5 * _B


def _sc_segment_sums(q, c1, c2, c3, px, py, pz, batch):
    mesh = plsc.VectorSubcoreMesh(core_axis_name="c", subcore_axis_name="s")

    @functools.partial(
        pl.kernel,
        out_type=jax.ShapeDtypeStruct((_NW, _ACC), jnp.float32),
        mesh=mesh,
        scratch_types=[
            pltpu.VMEM((_ACC,), jnp.float32),
            pltpu.VMEM((8, _CH), jnp.float32),
            pltpu.VMEM((_CH,), jnp.int32),
        ],
        compiler_params=pltpu.CompilerParams(needs_layout_passes=False,
                                             use_tc_tiling_on_sc=False),
    )
    def body(q_h, c1_h, c2_h, c3_h, px_h, py_h, pz_h, b_h, out_hbm,
             acc, colb, bb):
        c = lax.axis_index("c")
        s = lax.axis_index("s")
        wid = s * _NC + c
        base_atom = wid * _W
        iota = lax.iota(jnp.int32, 16)
        zeros16f = jnp.zeros((16,), jnp.float32)

        def zero_chunk(i, carry):
            acc[pl.ds(i * 16, 16)] = zeros16f
            return carry
        lax.fori_loop(0, _ACC // 16, zero_chunk, 0)

        def vec_body(vi, carry):
            b0 = vi * 16
            ids = bb[pl.ds(b0, 16)]
            q_v = colb[0, pl.ds(b0, 16)]
            c1_v = colb[1, pl.ds(b0, 16)]
            c2_v = colb[2, pl.ds(b0, 16)]
            c3_v = colb[3, pl.ds(b0, 16)]
            px_v = colb[4, pl.ds(b0, 16)]
            py_v = colb[5, pl.ds(b0, 16)]
            pz_v = colb[6, pl.ds(b0, 16)]

            d0 = q_v * px_v + c3_v
            d1 = q_v * py_v + c1_v
            d2 = q_v * pz_v + c2_v
            r2 = px_v * px_v + py_v * py_v + pz_v * pz_v
            pdr = px_v * c3_v + py_v * c1_v + pz_v * c2_v
            qq = r2 * q_v + 2.0 * pdr

            nxt = b0 + jnp.minimum(iota + 1, 15)
            ids_n = plsc.load_gather(bb, [nxt])
            is_bound = ids != ids_n
            is_last = jnp.logical_or(is_bound, iota == 15)

            for k, contrib in enumerate((q_v, d0, d1, d2, qq)):
                cum = plsc.cumsum(contrib)
                off = jnp.int32(k * _B)
                plsc.addupdate_scatter(acc, [ids + off], cum, mask=is_last)
                plsc.addupdate_scatter(acc, [ids_n + off], -cum,
                                       mask=is_bound)
            return carry

        def chunk_body(ci, carry):
            start = base_atom + ci * _CH
            sl = pl.ds(start, _CH)
            for j, col in enumerate((q_h, c1_h, c2_h, c3_h, px_h, py_h,
                                     pz_h)):
                pltpu.sync_copy(col.at[sl], colb.at[j])
            pltpu.sync_copy(b_h.at[sl], bb)
            return lax.fori_loop(0, _NVEC, vec_body, carry)

        lax.fori_loop(0, _NCHUNK, chunk_body, 0)
        pltpu.sync_copy(acc, out_hbm.at[wid])

    return body(q, c1, c2, c3, px, py, pz, batch)


def _tc_combine_body(p_ref, v_ref, o_ref):
    p = jnp.sum(p_ref[...], axis=0)
    t = p[0 * _B:1 * _B]
    d0 = p[1 * _B:2 * _B]
    d1 = p[2 * _B:3 * _B]
    d2 = p[3 * _B:4 * _B]
    quad = p[4 * _B:5 * _B]
    vol = v_ref[...]
    ls = jnp.power(vol, 0.3333)
    de = 0.5 * _CUBIC_MADELUNG * _CONST * t * t / ls
    de = de + 2.0 * _CONST * _PI * (d0 * d0 + d1 * d1 + d2 * d2) / (3.0 * vol)
    de = de - 2.0 * _CONST * _PI * t * quad / (3.0 * vol)
    o_ref[...] = de


def kernel(charge_coefficients, positions, volumes, batch):
    batch_i = batch.astype(jnp.int32)
    cols = [charge_coefficients[:, j] for j in range(4)]
    pcols = [positions[:, j] for j in range(3)]
    partials = _sc_segment_sums(*cols, *pcols, batch_i)
    de = pl.pallas_call(
        _tc_combine_body,
        out_shape=jax.ShapeDtypeStruct((_B,), jnp.float32),
    )(partials, volumes)
    return de


# speedup vs baseline: 29.8612x; 1.4745x over previous
import functools
import math

import jax
import jax.numpy as jnp
from jax import lax
from jax.experimental import pallas as pl
from jax.experimental.pallas import tpu as pltpu
from jax.experimental.pallas import tpu_sc as plsc

_FIELD_CONSTANT = 4.0 * math.pi * 14.399645351950548
_CUBIC_MADELUNG = -2.8372974794806
_CONST = _FIELD_CONSTANT / (4.0 * math.pi)
_PI = math.pi

_N = 1600000
_B = 4096
_NC = 2
_NS = 16
_NW = _NC * _NS
_W = _N // _NW
_CH = 2000
_NCHUNK = _W // _CH
_NVEC = _CH // 16
_ACC = 5 * _B


def _sc_segment_sums(q, c1, c2, c3, px, py, pz, batch):
    mesh = plsc.VectorSubcoreMesh(core_axis_name="c", subcore_axis_name="s")

    @functools.partial(
        pl.kernel,
        out_type=jax.ShapeDtypeStruct((_NW, _ACC), jnp.float32),
        mesh=mesh,
        scratch_types=[
            pltpu.VMEM((_ACC,), jnp.float32),
            pltpu.VMEM((2, 7, _CH), jnp.float32),
            pltpu.VMEM((2, _CH), jnp.int32),
            pltpu.SemaphoreType.DMA,
            pltpu.SemaphoreType.DMA,
        ],
        compiler_params=pltpu.CompilerParams(needs_layout_passes=False,
                                             use_tc_tiling_on_sc=False),
    )
    def body(q_h, c1_h, c2_h, c3_h, px_h, py_h, pz_h, b_h, out_hbm,
             acc, colb, bb, sem0, sem1):
        c = lax.axis_index("c")
        s = lax.axis_index("s")
        wid = s * _NC + c
        base_atom = wid * _W
        iota = lax.iota(jnp.int32, 16)
        zeros16f = jnp.zeros((16,), jnp.float32)
        cols = (q_h, c1_h, c2_h, c3_h, px_h, py_h, pz_h)
        sems = (sem0, sem1)

        def zero_chunk(i, carry):
            acc[pl.ds(i * 16, 16)] = zeros16f
            return carry
        lax.fori_loop(0, _ACC // 16, zero_chunk, 0)

        def fire(ci, b):
            sl = pl.ds(base_atom + ci * _CH, _CH)
            for j, col in enumerate(cols):
                pltpu.async_copy(col.at[sl], colb.at[b, j], sems[b])
            pltpu.async_copy(b_h.at[sl], bb.at[b], sems[b])

        def drain(b):
            for j, col in enumerate(cols):
                pltpu.make_async_copy(col.at[pl.ds(0, _CH)], colb.at[b, j],
                                      sems[b]).wait()
            pltpu.make_async_copy(b_h.at[pl.ds(0, _CH)], bb.at[b],
                                  sems[b]).wait()

        def vec_body_for(b):
            def vec_body(vi, carry):
                b0 = vi * 16
                ids = bb[b, pl.ds(b0, 16)]
                q_v = colb[b, 0, pl.ds(b0, 16)]
                c1_v = colb[b, 1, pl.ds(b0, 16)]
                c2_v = colb[b, 2, pl.ds(b0, 16)]
                c3_v = colb[b, 3, pl.ds(b0, 16)]
                px_v = colb[b, 4, pl.ds(b0, 16)]
                py_v = colb[b, 5, pl.ds(b0, 16)]
                pz_v = colb[b, 6, pl.ds(b0, 16)]

                d0 = q_v * px_v + c3_v
                d1 = q_v * py_v + c1_v
                d2 = q_v * pz_v + c2_v
                r2 = px_v * px_v + py_v * py_v + pz_v * pz_v
                pdr = px_v * c3_v + py_v * c1_v + pz_v * c2_v
                qq = r2 * q_v + 2.0 * pdr

                nxt = b0 + jnp.minimum(iota + 1, 15)
                ids_n = plsc.load_gather(bb.at[b], [nxt])
                is_bound = ids != ids_n
                is_last = jnp.logical_or(is_bound, iota == 15)

                for k, contrib in enumerate((q_v, d0, d1, d2, qq)):
                    cum = plsc.cumsum(contrib)
                    off = jnp.int32(k * _B)
                    plsc.addupdate_scatter(acc, [ids + off], cum,
                                           mask=is_last)
                    plsc.addupdate_scatter(acc, [ids_n + off], -cum,
                                           mask=is_bound)
                return carry
            return vec_body

        fire(0, 0)

        def pair_body(gi, carry):
            c0 = 2 * gi
            fire(c0 + 1, 1)
            drain(0)
            carry = lax.fori_loop(0, _NVEC, vec_body_for(0), carry)
            fire(c0 + 2, 0)
            drain(1)
            return lax.fori_loop(0, _NVEC, vec_body_for(1), carry)

        r = lax.fori_loop(0, (_NCHUNK - 1) // 2, pair_body, 0)
        drain(0)
        lax.fori_loop(0, _NVEC, vec_body_for(0), r)

        pltpu.sync_copy(acc, out_hbm.at[wid])

    return body(q, c1, c2, c3, px, py, pz, batch)


def _tc_combine_body(p_ref, v_ref, o_ref):
    p = jnp.sum(p_ref[...], axis=0)
    t = p[0 * _B:1 * _B]
    d0 = p[1 * _B:2 * _B]
    d1 = p[2 * _B:3 * _B]
    d2 = p[3 * _B:4 * _B]
    quad = p[4 * _B:5 * _B]
    vol = v_ref[...]
    ls = jnp.power(vol, 0.3333)
    de = 0.5 * _CUBIC_MADELUNG * _CONST * t * t / ls
    de = de + 2.0 * _CONST * _PI * (d0 * d0 + d1 * d1 + d2 * d2) / (3.0 * vol)
    de = de - 2.0 * _CONST * _PI * t * quad / (3.0 * vol)
    o_ref[...] = de


def kernel(charge_coefficients, positions, volumes, batch):
    batch_i = batch.astype(jnp.int32)
    cols = [charge_coefficients[:, j] for j in range(4)]
    pcols = [positions[:, j] for j in range(3)]
    partials = _sc_segment_sums(*cols, *pcols, batch_i)
    de = pl.pallas_call(
        _tc_combine_body,
        out_shape=jax.ShapeDtypeStruct((_B,), jnp.float32),
    )(partials, volumes)
    return de


# speedup vs baseline: 37.7749x; 1.2650x over previous
import functools
import math

import jax
import jax.numpy as jnp
from jax import lax
from jax.experimental import pallas as pl
from jax.experimental.pallas import tpu as pltpu
from jax.experimental.pallas import tpu_sc as plsc

_FIELD_CONSTANT = 4.0 * math.pi * 14.399645351950548
_CUBIC_MADELUNG = -2.8372974794806
_CONST = _FIELD_CONSTANT / (4.0 * math.pi)
_PI = math.pi

_N = 1600000
_B = 4096
_NC = 2
_NS = 16
_NW = _NC * _NS
_CH = 2000
_NVEC = _CH // 16
_ACC = 5 * _B
_PART_CHUNKS = (7, 6, 6, 6)


def _make_sc_call(nchunk):
    w = nchunk * _CH
    mesh = plsc.VectorSubcoreMesh(core_axis_name="c", subcore_axis_name="s")

    @functools.partial(
        pl.kernel,
        out_type=jax.ShapeDtypeStruct((_NW, _ACC), jnp.float32),
        mesh=mesh,
        scratch_types=[
            pltpu.VMEM((_ACC,), jnp.float32),
            pltpu.VMEM((2, 7, _CH), jnp.float32),
            pltpu.VMEM((2, _CH), jnp.int32),
            pltpu.SemaphoreType.DMA,
            pltpu.SemaphoreType.DMA,
        ],
        compiler_params=pltpu.CompilerParams(needs_layout_passes=False,
                                             use_tc_tiling_on_sc=False),
    )
    def body(q_h, c1_h, c2_h, c3_h, px_h, py_h, pz_h, b_h, out_hbm,
             acc, colb, bb, sem0, sem1):
        c = lax.axis_index("c")
        s = lax.axis_index("s")
        wid = s * _NC + c
        base_atom = wid * w
        iota = lax.iota(jnp.int32, 16)
        zeros16f = jnp.zeros((16,), jnp.float32)
        cols = (q_h, c1_h, c2_h, c3_h, px_h, py_h, pz_h)
        sems = (sem0, sem1)

        def zero_chunk(i, carry):
            acc[pl.ds(i * 16, 16)] = zeros16f
            return carry
        lax.fori_loop(0, _ACC // 16, zero_chunk, 0)

        def fire(ci, b):
            sl = pl.ds(base_atom + ci * _CH, _CH)
            for j, col in enumerate(cols):
                pltpu.async_copy(col.at[sl], colb.at[b, j], sems[b])
            pltpu.async_copy(b_h.at[sl], bb.at[b], sems[b])

        def drain(b):
            for j, col in enumerate(cols):
                pltpu.make_async_copy(col.at[pl.ds(0, _CH)], colb.at[b, j],
                                      sems[b]).wait()
            pltpu.make_async_copy(b_h.at[pl.ds(0, _CH)], bb.at[b],
                                  sems[b]).wait()

        def vec_body_for(b):
            def vec_body(vi, carry):
                b0 = vi * 16
                ids = bb[b, pl.ds(b0, 16)]
                q_v = colb[b, 0, pl.ds(b0, 16)]
                c1_v = colb[b, 1, pl.ds(b0, 16)]
                c2_v = colb[b, 2, pl.ds(b0, 16)]
                c3_v = colb[b, 3, pl.ds(b0, 16)]
                px_v = colb[b, 4, pl.ds(b0, 16)]
                py_v = colb[b, 5, pl.ds(b0, 16)]
                pz_v = colb[b, 6, pl.ds(b0, 16)]

                d0 = q_v * px_v + c3_v
                d1 = q_v * py_v + c1_v
                d2 = q_v * pz_v + c2_v
                r2 = px_v * px_v + py_v * py_v + pz_v * pz_v
                pdr = px_v * c3_v + py_v * c1_v + pz_v * c2_v
                qq = r2 * q_v + 2.0 * pdr

                nxt = b0 + jnp.minimum(iota + 1, 15)
                ids_n = plsc.load_gather(bb.at[b], [nxt])
                is_bound = ids != ids_n
                is_last = jnp.logical_or(is_bound, iota == 15)

                for k, contrib in enumerate((q_v, d0, d1, d2, qq)):
                    cum = plsc.cumsum(contrib)
                    off = jnp.int32(k * _B)
                    plsc.addupdate_scatter(acc, [ids + off], cum,
                                           mask=is_last)
                    plsc.addupdate_scatter(acc, [ids_n + off], -cum,
                                           mask=is_bound)
                return carry
            return vec_body

        fire(0, 0)

        def pair_body(gi, carry):
            c0 = 2 * gi
            fire(c0 + 1, 1)
            drain(0)
            carry = lax.fori_loop(0, _NVEC, vec_body_for(0), carry)

            @pl.when(c0 + 2 < nchunk)
            def _():
                fire(c0 + 2, 0)
            drain(1)
            return lax.fori_loop(0, _NVEC, vec_body_for(1), carry)

        r = lax.fori_loop(0, nchunk // 2, pair_body, 0)
        if nchunk % 2:
            drain(0)
            lax.fori_loop(0, _NVEC, vec_body_for(0), r)

        pltpu.sync_copy(acc, out_hbm.at[wid])

    return body


def _tc_combine_body(p_ref, v_ref, o_ref):
    p = jnp.sum(p_ref[...], axis=0)
    t = p[0 * _B:1 * _B]
    d0 = p[1 * _B:2 * _B]
    d1 = p[2 * _B:3 * _B]
    d2 = p[3 * _B:4 * _B]
    quad = p[4 * _B:5 * _B]
    vol = v_ref[...]
    ls = jnp.power(vol, 0.3333)
    de = 0.5 * _CUBIC_MADELUNG * _CONST * t * t / ls
    de = de + 2.0 * _CONST * _PI * (d0 * d0 + d1 * d1 + d2 * d2) / (3.0 * vol)
    de = de - 2.0 * _CONST * _PI * t * quad / (3.0 * vol)
    o_ref[...] = de


def kernel(charge_coefficients, positions, volumes, batch):
    batch_i = batch.astype(jnp.int32)
    partials = []
    start = 0
    for nchunk in _PART_CHUNKS:
        size = nchunk * _CH * _NW
        sl = slice(start, start + size)
        cols = [charge_coefficients[sl, j] for j in range(4)]
        pcols = [positions[sl, j] for j in range(3)]
        partials.append(_make_sc_call(nchunk)(*cols, *pcols, batch_i[sl]))
        start += size
    all_partials = jnp.concatenate(partials, axis=0)
    de = pl.pallas_call(
        _tc_combine_body,
        out_shape=jax.ShapeDtypeStruct((_B,), jnp.float32),
    )(all_partials, volumes)
    return de


# speedup vs baseline: 39.5828x; 1.0479x over previous
import functools
import math

import jax
import jax.numpy as jnp
from jax import lax
from jax.experimental import pallas as pl
from jax.experimental.pallas import tpu as pltpu
from jax.experimental.pallas import tpu_sc as plsc

_FIELD_CONSTANT = 4.0 * math.pi * 14.399645351950548
_CUBIC_MADELUNG = -2.8372974794806
_CONST = _FIELD_CONSTANT / (4.0 * math.pi)
_PI = math.pi

_N = 1600000
_B = 4096
_NC = 2
_NS = 16
_NW = _NC * _NS
_ACC = 5 * _B

_PART_BLOCKS = 4096
_NPARTS = 3
_WBLK = _PART_BLOCKS // _NW
_CHB = 16
_NCHUNK = _WBLK // _CHB
_CHA = _CHB * 128
_TAIL_START = _NPARTS * _PART_BLOCKS * 128
_TAIL_ATOMS = _N - _TAIL_START
_TAIL_W = _TAIL_ATOMS // _NW
_TAIL_VEC = _TAIL_W // 16


def _vec_step(acc, iota, ids, ids_n, q_v, c1_v, c2_v, c3_v, px_v, py_v, pz_v):
    d0 = q_v * px_v + c3_v
    d1 = q_v * py_v + c1_v
    d2 = q_v * pz_v + c2_v
    r2 = px_v * px_v + py_v * py_v + pz_v * pz_v
    pdr = px_v * c3_v + py_v * c1_v + pz_v * c2_v
    qq = r2 * q_v + 2.0 * pdr
    is_bound = ids != ids_n
    is_last = jnp.logical_or(is_bound, iota == 15)
    for k, contrib in enumerate((q_v, d0, d1, d2, qq)):
        cum = plsc.cumsum(contrib)
        off = jnp.int32(k * _B)
        plsc.addupdate_scatter(acc, [ids + off], cum, mask=is_last)
        plsc.addupdate_scatter(acc, [ids_n + off], -cum, mask=is_bound)


def _zero_acc(acc):
    zeros16f = jnp.zeros((16,), jnp.float32)

    def zero_chunk(i, carry):
        acc[pl.ds(i * 16, 16)] = zeros16f
        return carry
    lax.fori_loop(0, _ACC // 16, zero_chunk, 0)


_MESH = plsc.VectorSubcoreMesh(core_axis_name="c", subcore_axis_name="s")
_PARAMS = pltpu.CompilerParams(needs_layout_passes=False,
                               use_tc_tiling_on_sc=False)


def _make_main_call():
    @functools.partial(
        pl.kernel,
        out_type=jax.ShapeDtypeStruct((_NW, _ACC), jnp.float32),
        mesh=_MESH,
        scratch_types=[
            pltpu.VMEM((_ACC,), jnp.float32),
            pltpu.VMEM((2, _CHB * 512), jnp.float32),
            pltpu.VMEM((2, 3, _CHA), jnp.float32),
            pltpu.VMEM((2, _CHA), jnp.int32),
            pltpu.SemaphoreType.DMA,
            pltpu.SemaphoreType.DMA,
        ],
        compiler_params=_PARAMS,
    )
    def body(ccf_h, px_h, py_h, pz_h, b_h, out_hbm,
             acc, ccb, pcb, bb, sem0, sem1):
        c = lax.axis_index("c")
        s = lax.axis_index("s")
        wid = s * _NC + c
        base_blk = wid * _WBLK
        base_atom = base_blk * 128
        iota = lax.iota(jnp.int32, 16)
        sems = (sem0, sem1)
        pcols = (px_h, py_h, pz_h)

        _zero_acc(acc)

        def fire(ci, b):
            pltpu.async_copy(
                ccf_h.at[pl.ds((base_blk + ci * _CHB) * 512, _CHB * 512)],
                ccb.at[b], sems[b])
            sl = pl.ds(base_atom + ci * _CHA, _CHA)
            for j, col in enumerate(pcols):
                pltpu.async_copy(col.at[sl], pcb.at[b, j], sems[b])
            pltpu.async_copy(b_h.at[sl], bb.at[b], sems[b])

        def drain(b):
            pltpu.make_async_copy(ccf_h.at[pl.ds(0, _CHB * 512)], ccb.at[b],
                                  sems[b]).wait()
            for j, col in enumerate(pcols):
                pltpu.make_async_copy(col.at[pl.ds(0, _CHA)], pcb.at[b, j],
                                      sems[b]).wait()
            pltpu.make_async_copy(b_h.at[pl.ds(0, _CHA)], bb.at[b],
                                  sems[b]).wait()

        def vec_body_for(b):
            def vec_body(vi, carry):
                a0 = vi * 16
                cbase = lax.shift_right_logical(vi, 3) * 512 \
                    + lax.bitwise_and(vi, 7) * 16
                ids = bb[b, pl.ds(a0, 16)]
                q_v = ccb[b, pl.ds(cbase, 16)]
                c1_v = ccb[b, pl.ds(cbase + 128, 16)]
                c2_v = ccb[b, pl.ds(cbase + 256, 16)]
                c3_v = ccb[b, pl.ds(cbase + 384, 16)]
                px_v = pcb[b, 0, pl.ds(a0, 16)]
                py_v = pcb[b, 1, pl.ds(a0, 16)]
                pz_v = pcb[b, 2, pl.ds(a0, 16)]
                nxt = a0 + jnp.minimum(iota + 1, 15)
                ids_n = plsc.load_gather(bb.at[b], [nxt])
                _vec_step(acc, iota, ids, ids_n,
                          q_v, c1_v, c2_v, c3_v, px_v, py_v, pz_v)
                return carry
            return vec_body

        nvec = _CHA // 16
        fire(0, 0)

        def pair_body(gi, carry):
            c0 = 2 * gi
            fire(c0 + 1, 1)
            drain(0)
            carry = lax.fori_loop(0, nvec, vec_body_for(0), carry)

            @pl.when(c0 + 2 < _NCHUNK)
            def _():
                fire(c0 + 2, 0)
            drain(1)
            return lax.fori_loop(0, nvec, vec_body_for(1), carry)

        lax.fori_loop(0, _NCHUNK // 2, pair_body, 0)
        pltpu.sync_copy(acc, out_hbm.at[wid])

    return body


def _make_tail_call():
    @functools.partial(
        pl.kernel,
        out_type=jax.ShapeDtypeStruct((_NW, _ACC), jnp.float32),
        mesh=_MESH,
        scratch_types=[
            pltpu.VMEM((_ACC,), jnp.float32),
            pltpu.VMEM((7, _TAIL_W), jnp.float32),
            pltpu.VMEM((_TAIL_W,), jnp.int32),
        ],
        compiler_params=_PARAMS,
    )
    def body(q_h, c1_h, c2_h, c3_h, px_h, py_h, pz_h, b_h, out_hbm,
             acc, colb, bb):
        c = lax.axis_index("c")
        s = lax.axis_index("s")
        wid = s * _NC + c
        base = wid * _TAIL_W
        iota = lax.iota(jnp.int32, 16)

        _zero_acc(acc)
        sl = pl.ds(base, _TAIL_W)
        for j, col in enumerate((q_h, c1_h, c2_h, c3_h, px_h, py_h, pz_h)):
            pltpu.sync_copy(col.at[sl], colb.at[j])
        pltpu.sync_copy(b_h.at[sl], bb)

        def vec_body(vi, carry):
            a0 = vi * 16
            ids = bb[pl.ds(a0, 16)]
            q_v = colb[0, pl.ds(a0, 16)]
            c1_v = colb[1, pl.ds(a0, 16)]
            c2_v = colb[2, pl.ds(a0, 16)]
            c3_v = colb[3, pl.ds(a0, 16)]
            px_v = colb[4, pl.ds(a0, 16)]
            py_v = colb[5, pl.ds(a0, 16)]
            pz_v = colb[6, pl.ds(a0, 16)]
            nxt = a0 + jnp.minimum(iota + 1, 15)
            ids_n = plsc.load_gather(bb, [nxt])
            _vec_step(acc, iota, ids, ids_n,
                      q_v, c1_v, c2_v, c3_v, px_v, py_v, pz_v)
            return carry

        lax.fori_loop(0, _TAIL_VEC, vec_body, 0)
        pltpu.sync_copy(acc, out_hbm.at[wid])

    return body


def _tc_combine_body(p_ref, v_ref, o_ref):
    p = jnp.sum(p_ref[...], axis=0)
    t = p[0 * _B:1 * _B]
    d0 = p[1 * _B:2 * _B]
    d1 = p[2 * _B:3 * _B]
    d2 = p[3 * _B:4 * _B]
    quad = p[4 * _B:5 * _B]
    vol = v_ref[...]
    ls = jnp.power(vol, 0.3333)
    de = 0.5 * _CUBIC_MADELUNG * _CONST * t * t / ls
    de = de + 2.0 * _CONST * _PI * (d0 * d0 + d1 * d1 + d2 * d2) / (3.0 * vol)
    de = de - 2.0 * _CONST * _PI * t * quad / (3.0 * vol)
    o_ref[...] = de


def kernel(charge_coefficients, positions, volumes, batch):
    batch_i = batch.astype(jnp.int32)
    partials = []
    for p in range(_NPARTS):
        s = p * _PART_BLOCKS * 128
        e = s + _PART_BLOCKS * 128
        ccf = (charge_coefficients[s:e]
               .reshape(_PART_BLOCKS, 128, 4)
               .transpose(0, 2, 1)
               .reshape(_PART_BLOCKS * 512))
        pcols = [positions[s:e, j] for j in range(3)]
        partials.append(
            _make_main_call()(ccf, *pcols, batch_i[s:e]))
    ts = _TAIL_START
    tcols = [charge_coefficients[ts:, j] for j in range(4)]
    tpcols = [positions[ts:, j] for j in range(3)]
    partials.append(_make_tail_call()(*tcols, *tpcols, batch_i[ts:]))
    all_partials = jnp.concatenate(partials, axis=0)
    de = pl.pallas_call(
        _tc_combine_body,
        out_shape=jax.ShapeDtypeStruct((_B,), jnp.float32),
    )(all_partials, volumes)
    return de


# speedup vs baseline: 45.9832x; 1.1617x over previous
import functools
import math

import jax
import jax.numpy as jnp
from jax import lax
from jax.experimental import pallas as pl
from jax.experimental.pallas import tpu as pltpu
from jax.experimental.pallas import tpu_sc as plsc

_FIELD_CONSTANT = 4.0 * math.pi * 14.399645351950548
_CUBIC_MADELUNG = -2.8372974794806
_CONST = _FIELD_CONSTANT / (4.0 * math.pi)
_PI = math.pi

_N = 1600000
_B = 4096
_NC = 2
_NS = 16
_NW = _NC * _NS
_ACC = 5 * _B

_PART_BLOCKS = 4096
_NPARTS = 3
_WBLK = _PART_BLOCKS // _NW
_CHB = 16
_NCHUNK = _WBLK // _CHB
_CHA = _CHB * 128
_TAIL_START = _NPARTS * _PART_BLOCKS * 128
_TAIL_ATOMS = _N - _TAIL_START
_TAIL_W = _TAIL_ATOMS // _NW
_TAIL_VEC = _TAIL_W // 16


def _vec_step(acc, iota, ids, ids_n, q_v, c1_v, c2_v, c3_v, px_v, py_v, pz_v):
    d0 = q_v * px_v + c3_v
    d1 = q_v * py_v + c1_v
    d2 = q_v * pz_v + c2_v
    r2 = px_v * px_v + py_v * py_v + pz_v * pz_v
    pdr = px_v * c3_v + py_v * c1_v + pz_v * c2_v
    qq = r2 * q_v + 2.0 * pdr
    is_bound = ids != ids_n
    is_last = jnp.logical_or(is_bound, iota == 15)
    for k, contrib in enumerate((q_v, d0, d1, d2, qq)):
        cum = plsc.cumsum(contrib)
        off = jnp.int32(k * _B)
        plsc.addupdate_scatter(acc, [ids + off], cum, mask=is_last)
        plsc.addupdate_scatter(acc, [ids_n + off], -cum, mask=is_bound)


def _zero_acc(acc):
    zeros16f = jnp.zeros((16,), jnp.float32)

    def zero_chunk(i, carry):
        acc[pl.ds(i * 16, 16)] = zeros16f
        return carry
    lax.fori_loop(0, _ACC // 16, zero_chunk, 0)


_MESH = plsc.VectorSubcoreMesh(core_axis_name="c", subcore_axis_name="s")
_PARAMS = pltpu.CompilerParams(needs_layout_passes=False,
                               use_tc_tiling_on_sc=False)


def _make_main_call():
    @functools.partial(
        pl.kernel,
        out_type=jax.ShapeDtypeStruct((_NW, _ACC), jnp.float32),
        mesh=_MESH,
        scratch_types=[
            pltpu.VMEM((_ACC,), jnp.float32),
            pltpu.VMEM((2, _CHB * 512), jnp.float32),
            pltpu.VMEM((2, 3, _CHA), jnp.float32),
            pltpu.VMEM((2, _CHA), jnp.int32),
            pltpu.SemaphoreType.DMA,
            pltpu.SemaphoreType.DMA,
        ],
        compiler_params=_PARAMS,
    )
    def body(ccf_h, px_h, py_h, pz_h, b_h, out_hbm,
             acc, ccb, pcb, bb, sem0, sem1):
        c = lax.axis_index("c")
        s = lax.axis_index("s")
        wid = s * _NC + c
        base_blk = wid * _WBLK
        base_atom = base_blk * 128
        iota = lax.iota(jnp.int32, 16)
        sems = (sem0, sem1)
        pcols = (px_h, py_h, pz_h)

        _zero_acc(acc)

        def fire(ci, b):
            pltpu.async_copy(
                ccf_h.at[pl.ds((base_blk + ci * _CHB) * 512, _CHB * 512)],
                ccb.at[b], sems[b])
            sl = pl.ds(base_atom + ci * _CHA, _CHA)
            for j, col in enumerate(pcols):
                pltpu.async_copy(col.at[sl], pcb.at[b, j], sems[b])
            pltpu.async_copy(b_h.at[sl], bb.at[b], sems[b])

        def drain(b):
            pltpu.make_async_copy(ccf_h.at[pl.ds(0, _CHB * 512)], ccb.at[b],
                                  sems[b]).wait()
            for j, col in enumerate(pcols):
                pltpu.make_async_copy(col.at[pl.ds(0, _CHA)], pcb.at[b, j],
                                      sems[b]).wait()
            pltpu.make_async_copy(b_h.at[pl.ds(0, _CHA)], bb.at[b],
                                  sems[b]).wait()

        def vec_body_for(b):
            def vec_body(vi, carry):
                a0 = vi * 16
                cbase = lax.shift_right_logical(vi, 3) * 512 \
                    + lax.bitwise_and(vi, 7) * 16
                ids = bb[b, pl.ds(a0, 16)]
                q_v = ccb[b, pl.ds(cbase, 16)]
                c1_v = ccb[b, pl.ds(cbase + 128, 16)]
                c2_v = ccb[b, pl.ds(cbase + 256, 16)]
                c3_v = ccb[b, pl.ds(cbase + 384, 16)]
                px_v = pcb[b, 0, pl.ds(a0, 16)]
                py_v = pcb[b, 1, pl.ds(a0, 16)]
                pz_v = pcb[b, 2, pl.ds(a0, 16)]
                nxt = a0 + jnp.minimum(iota + 1, 15)
                ids_n = plsc.load_gather(bb.at[b], [nxt])
                _vec_step(acc, iota, ids, ids_n,
                          q_v, c1_v, c2_v, c3_v, px_v, py_v, pz_v)
                return carry
            return vec_body

        nvec = _CHA // 16

        def compute(b, carry):
            return plsc.parallel_loop(0, nvec, 1, unroll=2,
                                      carry=carry)(vec_body_for(b))

        fire(0, 0)

        def pair_body(gi, carry):
            c0 = 2 * gi
            fire(c0 + 1, 1)
            drain(0)
            carry = compute(0, carry)

            @pl.when(c0 + 2 < _NCHUNK)
            def _():
                fire(c0 + 2, 0)
            drain(1)
            return compute(1, carry)

        lax.fori_loop(0, _NCHUNK // 2, pair_body, 0)
        pltpu.sync_copy(acc, out_hbm.at[wid])

    return body


def _make_tail_call():
    @functools.partial(
        pl.kernel,
        out_type=jax.ShapeDtypeStruct((_NW, _ACC), jnp.float32),
        mesh=_MESH,
        scratch_types=[
            pltpu.VMEM((_ACC,), jnp.float32),
            pltpu.VMEM((7, _TAIL_W), jnp.float32),
            pltpu.VMEM((_TAIL_W,), jnp.int32),
        ],
        compiler_params=_PARAMS,
    )
    def body(q_h, c1_h, c2_h, c3_h, px_h, py_h, pz_h, b_h, out_hbm,
             acc, colb, bb):
        c = lax.axis_index("c")
        s = lax.axis_index("s")
        wid = s * _NC + c
        base = wid * _TAIL_W
        iota = lax.iota(jnp.int32, 16)

        _zero_acc(acc)
        sl = pl.ds(base, _TAIL_W)
        for j, col in enumerate((q_h, c1_h, c2_h, c3_h, px_h, py_h, pz_h)):
            pltpu.sync_copy(col.at[sl], colb.at[j])
        pltpu.sync_copy(b_h.at[sl], bb)

        def vec_body(vi, carry):
            a0 = vi * 16
            ids = bb[pl.ds(a0, 16)]
            q_v = colb[0, pl.ds(a0, 16)]
            c1_v = colb[1, pl.ds(a0, 16)]
            c2_v = colb[2, pl.ds(a0, 16)]
            c3_v = colb[3, pl.ds(a0, 16)]
            px_v = colb[4, pl.ds(a0, 16)]
            py_v = colb[5, pl.ds(a0, 16)]
            pz_v = colb[6, pl.ds(a0, 16)]
            nxt = a0 + jnp.minimum(iota + 1, 15)
            ids_n = plsc.load_gather(bb, [nxt])
            _vec_step(acc, iota, ids, ids_n,
                      q_v, c1_v, c2_v, c3_v, px_v, py_v, pz_v)
            return carry

        lax.fori_loop(0, _TAIL_VEC, vec_body, 0)
        pltpu.sync_copy(acc, out_hbm.at[wid])

    return body


def _tc_combine_body(p_ref, v_ref, o_ref):
    p = jnp.sum(p_ref[...], axis=0)
    t = p[0 * _B:1 * _B]
    d0 = p[1 * _B:2 * _B]
    d1 = p[2 * _B:3 * _B]
    d2 = p[3 * _B:4 * _B]
    quad = p[4 * _B:5 * _B]
    vol = v_ref[...]
    ls = jnp.power(vol, 0.3333)
    de = 0.5 * _CUBIC_MADELUNG * _CONST * t * t / ls
    de = de + 2.0 * _CONST * _PI * (d0 * d0 + d1 * d1 + d2 * d2) / (3.0 * vol)
    de = de - 2.0 * _CONST * _PI * t * quad / (3.0 * vol)
    o_ref[...] = de


def kernel(charge_coefficients, positions, volumes, batch):
    batch_i = batch.astype(jnp.int32)
    partials = []
    for p in range(_NPARTS):
        s = p * _PART_BLOCKS * 128
        e = s + _PART_BLOCKS * 128
        ccf = (charge_coefficients[s:e]
               .reshape(_PART_BLOCKS, 128, 4)
               .transpose(0, 2, 1)
               .reshape(_PART_BLOCKS * 512))
        pcols = [positions[s:e, j] for j in range(3)]
        partials.append(
            _make_main_call()(ccf, *pcols, batch_i[s:e]))
    ts = _TAIL_START
    tcols = [charge_coefficients[ts:, j] for j in range(4)]
    tpcols = [positions[ts:, j] for j in range(3)]
    partials.append(_make_tail_call()(*tcols, *tpcols, batch_i[ts:]))
    all_partials = jnp.concatenate(partials, axis=0)
    de = pl.pallas_call(
        _tc_combine_body,
        out_shape=jax.ShapeDtypeStruct((_B,), jnp.float32),
    )(all_partials, volumes)
    return de
